# Initial kernel scaffold; baseline (speedup 1.0000x reference)
import jax
import jax.numpy as jnp
from jax.experimental import pallas as pl


def kernel(token_ids, weight):
    raise NotImplementedError("write your pallas kernel here")


# speedup vs baseline: 1.8778x; 1.8778x over previous
import functools

import jax
import jax.numpy as jnp
from jax import lax
from jax.experimental import pallas as pl
from jax.experimental.pallas import tpu as pltpu
from jax.experimental.pallas import tpu_sc as plsc

NBUF = 4
C = 400


@functools.lru_cache(maxsize=None)
def _build_gather(B, D):
    info = plsc.get_sparse_core_info()
    NC, NS = info.num_cores, info.num_subcores
    NW = NC * NS
    assert B % NW == 0
    b_per_w = B // NW
    assert b_per_w % C == 0
    n = b_per_w // C
    assert n % NBUF == 0 and n >= 2 * NBUF
    mesh = plsc.VectorSubcoreMesh(core_axis_name="c", subcore_axis_name="s")

    @functools.partial(
        pl.kernel,
        mesh=mesh,
        out_type=jax.ShapeDtypeStruct((B, D), jnp.float32),
        scratch_types=[
            pltpu.VMEM((NBUF, C), jnp.int32),
            pltpu.VMEM((NBUF, C, D), jnp.float32),
            pltpu.SemaphoreType.DMA((NBUF,)),
            pltpu.SemaphoreType.DMA((NBUF,)),
        ],
        compiler_params=pltpu.CompilerParams(use_tc_tiling_on_sc=False),
    )
    def gather_kernel(idx_hbm, table_hbm, out_hbm, idx_v, rows_v, gsem, ssem):
        wid = lax.axis_index("s") * NC + lax.axis_index("c")
        base = wid * b_per_w

        def fire_gather(c, b):
            off = pl.multiple_of(base + c * C, C)
            pltpu.sync_copy(idx_hbm.at[pl.ds(off, C)], idx_v.at[b])
            pltpu.async_copy(table_hbm.at[idx_v.at[b]], rows_v.at[b], gsem.at[b])

        def fire_store(c, b):
            off = pl.multiple_of(base + c * C, C)
            pltpu.async_copy(rows_v.at[b], out_hbm.at[pl.ds(off, C)], ssem.at[b])

        def wait_gather(b):
            pltpu.make_async_copy(
                table_hbm.at[idx_v.at[b]], rows_v.at[b], gsem.at[b]
            ).wait()

        def wait_store(c, b):
            off = pl.multiple_of(base + c * C, C)
            pltpu.make_async_copy(
                rows_v.at[b], out_hbm.at[pl.ds(off, C)], ssem.at[b]
            ).wait()

        fire_gather(0, 0)
        fire_gather(1, 1)

        def outer(t, carry):
            for b in range(NBUF):
                i = t * NBUF + b

                @pl.when(i + 2 < n)
                def _(b=b, i=i):
                    q = (b + 2) % NBUF

                    @pl.when(i >= 2)
                    def _():
                        wait_store(i - 2, q)

                    fire_gather(i + 2, q)

                wait_gather(b)
                fire_store(i, b)
            return carry

        lax.fori_loop(0, n // NBUF, outer, 0)

        for b in range(NBUF):
            wait_store(n - NBUF + b, b)

    return gather_kernel


def kernel(token_ids, weight):
    Bt, T = token_ids.shape
    V, D = weight.shape
    B = Bt * T
    idx = token_ids.reshape(B).astype(jnp.int32)
    out = _build_gather(B, D)(idx, weight)
    return out.reshape(Bt, T, D)
